# baseline (speedup 1.0000x reference)
import dataclasses
import functools

import jax
import jax.numpy as jnp
from jax import lax
from jax.experimental import pallas as pl
from jax.experimental.pallas import tpu as pltpu
from jax.experimental.pallas import tpu_sc as plsc

N = 10000
E = 160000
D_IN = 256
HID = 256
NCLS = 64
HW = HID // 2

NC = 2
NS = 16

N_PAD = 10240
RCH = 128
RCH_SUB = N_PAD // NS // RCH

K_EDGE = 128
E_PAD = 163840
E_SUB = E_PAD // NS
CH_SUB = E_SUB // K_EDGE
ROUND = 40
NROUNDS = CH_SUB // ROUND


@functools.cache
def _get_sc_aggregate():
    mesh = plsc.VectorSubcoreMesh(core_axis_name="c", subcore_axis_name="s")
    cp = pltpu.CompilerParams()
    if "needs_layout_passes" in pltpu.CompilerParams.__dataclass_fields__:
        cp = dataclasses.replace(cp, needs_layout_passes=False)

    @functools.partial(
        pl.kernel,
        out_type=jax.ShapeDtypeStruct((NC, N_PAD, HW), jnp.float32),
        mesh=mesh,
        compiler_params=cp,
        scratch_types=[
            pltpu.VMEM((ROUND, K_EDGE), jnp.int32),
            pltpu.VMEM((ROUND, K_EDGE), jnp.int32),
            pltpu.VMEM((16,), jnp.int32),
            pltpu.VMEM((K_EDGE, HW), jnp.float32),
            pltpu.VMEM((K_EDGE, HW), jnp.float32),
            pltpu.VMEM_SHARED((N_PAD, HW), jnp.float32),
            pltpu.SemaphoreType.DMA,
            pltpu.SemaphoreType.DMA,
            pltpu.SemaphoreType.DMA,
            pltpu.SemaphoreType.DMA,
        ],
    )
    def sc_aggregate(g_hbm, src2_hbm, dst2_hbm, nrounds_hbm, out_hbm, sidx,
                     didx, flag_v, rows0, rows1, acc, gsem0, gsem1, ssem0,
                     ssem1):
        c = lax.axis_index("c")
        s = lax.axis_index("s")
        rows = (rows0, rows1)
        gsem = (gsem0, gsem1)
        ssem = (ssem0, ssem1)
        row0 = s * RCH_SUB * RCH

        st_fl = [None, None]
        for j in range(RCH_SUB):
            b = j & 1
            if st_fl[b] is not None:
                st_fl[b].wait()
            pltpu.async_copy(
                g_hbm.at[pl.ds(c * N_PAD + row0 + j * RCH, RCH)], rows[b],
                gsem[b]).wait()
            st_fl[b] = pltpu.async_copy(
                rows[b], acc.at[pl.ds(row0 + j * RCH, RCH)], ssem[b])
        for h in st_fl:
            h.wait()
        plsc.subcore_barrier()

        pltpu.sync_copy(nrounds_hbm, flag_v)
        nrounds = jnp.max(flag_v[...])

        def _round(r):
            row = c * (E_PAD // K_EDGE) + s * CH_SUB + r * ROUND
            pltpu.sync_copy(src2_hbm.at[pl.ds(row, ROUND)], sidx)
            pltpu.sync_copy(dst2_hbm.at[pl.ds(row, ROUND)], didx)
            g_fl = [None, None]
            s_fl = [None, None]
            for j in range(ROUND):
                b = j & 1
                if s_fl[b] is not None:
                    s_fl[b].wait()
                g_fl[b] = pltpu.async_copy(g_hbm.at[sidx.at[j]], rows[b],
                                           gsem[b])
                pb = 1 - b
                if g_fl[pb] is not None:
                    g_fl[pb].wait()
                    s_fl[pb] = pltpu.async_copy(
                        rows[pb], acc.at[didx.at[j - 1]], ssem[pb], add=True)
            lb = (ROUND - 1) & 1
            g_fl[lb].wait()
            s_fl[lb] = pltpu.async_copy(rows[lb], acc.at[didx.at[ROUND - 1]],
                                        ssem[lb], add=True)
            for h in s_fl:
                if h is not None:
                    h.wait()

        _round(0)

        @pl.when(nrounds >= 2)
        def _():
            _round(1)

        plsc.subcore_barrier()

        st_fl = [None, None]
        for j in range(RCH_SUB):
            b = j & 1
            if st_fl[b] is not None:
                st_fl[b].wait()
            pltpu.async_copy(acc.at[pl.ds(row0 + j * RCH, RCH)], rows[b],
                             gsem[b]).wait()
            st_fl[b] = pltpu.async_copy(
                rows[b], out_hbm.at[c, pl.ds(row0 + j * RCH, RCH)], ssem[b])
        for h in st_fl:
            h.wait()

    return sc_aggregate


R = 1024


def _dinv_from_cnt(cnt_blk):
    return lax.rsqrt(cnt_blk[0, :, :1] + cnt_blk[1, :, :1] - 1.0)


def _mm_first_body(x_ref, w_ref, cnt_ref, o_ref):
    dinv = _dinv_from_cnt(cnt_ref[...])
    g = jnp.dot(x_ref[...], w_ref[...],
                preferred_element_type=jnp.float32) * dinv
    o_ref[0] = g[:, :HW]
    o_ref[1] = g[:, HW:]


_mm_first = pl.pallas_call(
    _mm_first_body,
    grid=(N_PAD // R,),
    in_specs=[
        pl.BlockSpec((R, D_IN), lambda i: (i, 0)),
        pl.BlockSpec((D_IN, HID), lambda i: (0, 0)),
        pl.BlockSpec((NC, R, HW), lambda i: (0, i, 0)),
    ],
    out_specs=pl.BlockSpec((NC, R, HW), lambda i: (0, i, 0)),
    out_shape=jax.ShapeDtypeStruct((NC, N_PAD, HW), jnp.float32),
)


def _mm_mid_body(a_ref, cnt_ref, b_ref, w_ref, o_ref):
    dinv = _dinv_from_cnt(cnt_ref[...])
    aa = jnp.concatenate([a_ref[0], a_ref[1]], axis=1)
    h = jnp.maximum(aa * dinv + b_ref[...], 0.0)
    g = jnp.dot(h, w_ref[...], preferred_element_type=jnp.float32) * dinv
    o_ref[0] = g[:, :HW]
    o_ref[1] = g[:, HW:]


_mm_mid = pl.pallas_call(
    _mm_mid_body,
    grid=(N_PAD // R,),
    in_specs=[
        pl.BlockSpec((NC, R, HW), lambda i: (0, i, 0)),
        pl.BlockSpec((NC, R, HW), lambda i: (0, i, 0)),
        pl.BlockSpec((1, HID), lambda i: (0, 0)),
        pl.BlockSpec((HID, HID), lambda i: (0, 0)),
    ],
    out_specs=pl.BlockSpec((NC, R, HW), lambda i: (0, i, 0)),
    out_shape=jax.ShapeDtypeStruct((NC, N_PAD, HW), jnp.float32),
)


def _scale_mid_body(a_ref, cnt_ref, b_ref, o_ref):
    dinv = _dinv_from_cnt(cnt_ref[...])
    aa = jnp.concatenate([a_ref[0], a_ref[1]], axis=1)
    q = jnp.maximum(aa * dinv + b_ref[...], 0.0) * dinv
    o_ref[0] = q[:, :HW]
    o_ref[1] = q[:, HW:]


_scale_mid = pl.pallas_call(
    _scale_mid_body,
    grid=(N_PAD // R,),
    in_specs=[
        pl.BlockSpec((NC, R, HW), lambda i: (0, i, 0)),
        pl.BlockSpec((NC, R, HW), lambda i: (0, i, 0)),
        pl.BlockSpec((1, HID), lambda i: (0, 0)),
    ],
    out_specs=pl.BlockSpec((NC, R, HW), lambda i: (0, i, 0)),
    out_shape=jax.ShapeDtypeStruct((NC, N_PAD, HW), jnp.float32),
)


def _mm_last_body(a_ref, cnt_ref, b_ref, w_ref, o_ref):
    dinv = _dinv_from_cnt(cnt_ref[...])
    aa = jnp.concatenate([a_ref[0], a_ref[1]], axis=1)
    o_ref[...] = jnp.dot(aa * dinv, w_ref[...],
                         preferred_element_type=jnp.float32) + b_ref[...]


_mm_last = pl.pallas_call(
    _mm_last_body,
    grid=(N_PAD // R,),
    in_specs=[
        pl.BlockSpec((NC, R, HW), lambda i: (0, i, 0)),
        pl.BlockSpec((NC, R, HW), lambda i: (0, i, 0)),
        pl.BlockSpec((1, NCLS), lambda i: (0, 0)),
        pl.BlockSpec((HID, NCLS), lambda i: (0, 0)),
    ],
    out_specs=pl.BlockSpec((R, NCLS), lambda i: (i, 0)),
    out_shape=jax.ShapeDtypeStruct((N, NCLS), jnp.float32),
)


def kernel(x, edge_index, W1, b1, W2, b2, W3, b3):
    src = edge_index[0].astype(jnp.int32)
    dst = edge_index[1].astype(jnp.int32)
    pad = E_PAD - E
    src = jnp.concatenate([src, jnp.zeros((pad,), jnp.int32)])
    dst = jnp.concatenate([dst, jnp.full((pad,), N, jnp.int32)])
    srcs2 = jnp.concatenate([src, src + N_PAD]).reshape(-1, K_EDGE)
    dst2 = jnp.concatenate([dst, dst]).reshape(-1, K_EDGE)
    two = jnp.full((16,), 2, jnp.int32)

    agg = _get_sc_aggregate()

    e_half_sub = E_PAD // (NC * NS)
    core_off = jnp.array([0, N_PAD], jnp.int32)[:, None, None]
    src_h = src.reshape(NC, NS, e_half_sub) + core_off
    dst_h = dst.reshape(NC, NS, e_half_sub)
    cnt_src2 = jnp.concatenate(
        [src_h, jnp.zeros_like(src_h) + core_off], axis=2).reshape(-1, K_EDGE)
    cnt_dst2 = jnp.concatenate(
        [dst_h, jnp.full_like(dst_h, N)], axis=2).reshape(-1, K_EDGE)
    one = jnp.full((16,), 1, jnp.int32)
    cnt = agg(jnp.ones((NC * N_PAD, HW), jnp.float32), cnt_src2, cnt_dst2,
              one)

    g1 = _mm_first(x, W1, cnt)
    a1 = agg(g1.reshape(NC * N_PAD, HW), srcs2, dst2, two)

    g2 = _mm_mid(a1, cnt, b1.reshape(1, HID), W2)
    a2 = agg(g2.reshape(NC * N_PAD, HW), srcs2, dst2, two)

    q = _scale_mid(a2, cnt, b2.reshape(1, HID))
    a3 = agg(q.reshape(NC * N_PAD, HW), srcs2, dst2, two)

    return _mm_last(a3, cnt, b3.reshape(1, NCLS), W3)


# speedup vs baseline: 1.2863x; 1.2863x over previous
import functools

import jax
import jax.numpy as jnp
from jax import lax
from jax.experimental import pallas as pl
from jax.experimental.pallas import tpu as pltpu
from jax.experimental.pallas import tpu_sc as plsc

N = 10000
E = 160000
D_IN = 256
HID = 256
NCLS = 64
HW = HID // 2

NC = 2
NS = 16

N_PAD = 10240
RCH = 128
RCH_SUB = N_PAD // NS // RCH

K_EDGE = 128
E_PAD = 163840
E_SUB = E_PAD // NS
CH_SUB = E_SUB // K_EDGE
ROUND = 40
NROUNDS = CH_SUB // ROUND


@functools.cache
def _get_sc_aggregate():
    mesh = plsc.VectorSubcoreMesh(core_axis_name="c", subcore_axis_name="s")

    @functools.partial(
        pl.kernel,
        out_type=jax.ShapeDtypeStruct((NC, N_PAD, HW), jnp.float32),
        mesh=mesh,
        scratch_types=[
            pltpu.VMEM((ROUND, K_EDGE), jnp.int32),
            pltpu.VMEM((ROUND, K_EDGE), jnp.int32),
            pltpu.VMEM((K_EDGE, HW), jnp.float32),
            pltpu.VMEM((K_EDGE, HW), jnp.float32),
            pltpu.VMEM_SHARED((N_PAD, HW), jnp.float32),
            pltpu.SemaphoreType.DMA,
            pltpu.SemaphoreType.DMA,
            pltpu.SemaphoreType.DMA,
            pltpu.SemaphoreType.DMA,
        ],
    )
    def sc_aggregate(g_hbm, src2_hbm, dst2_hbm, out_hbm, sidx, didx, rows0,
                     rows1, acc, gsem0, gsem1, ssem0, ssem1):
        c = lax.axis_index("c")
        s = lax.axis_index("s")
        rows = (rows0, rows1)
        gsem = (gsem0, gsem1)
        ssem = (ssem0, ssem1)
        row0 = s * RCH_SUB * RCH

        st_fl = [None, None]
        for j in range(RCH_SUB):
            b = j & 1
            if st_fl[b] is not None:
                st_fl[b].wait()
            pltpu.async_copy(
                g_hbm.at[pl.ds(c * N_PAD + row0 + j * RCH, RCH)], rows[b],
                gsem[b]).wait()
            st_fl[b] = pltpu.async_copy(
                rows[b], acc.at[pl.ds(row0 + j * RCH, RCH)], ssem[b])
        for h in st_fl:
            h.wait()
        plsc.subcore_barrier()

        g_fl = [None, None]
        s_fl = [None, None]
        for r in range(NROUNDS):
            srow = c * (E_PAD // K_EDGE) + s * CH_SUB + r * ROUND
            drow = s * CH_SUB + r * ROUND
            pltpu.sync_copy(src2_hbm.at[pl.ds(srow, ROUND)], sidx)
            pltpu.sync_copy(dst2_hbm.at[pl.ds(drow, ROUND)], didx)
            for j in range(ROUND):
                b = j & 1
                if s_fl[b] is not None:
                    s_fl[b].wait()
                g_fl[b] = pltpu.async_copy(g_hbm.at[sidx.at[j]], rows[b],
                                           gsem[b])
                pb = 1 - b
                if g_fl[pb] is not None:
                    g_fl[pb].wait()
                    s_fl[pb] = pltpu.async_copy(
                        rows[pb], acc.at[didx.at[j - 1]], ssem[pb], add=True)
            lb = (ROUND - 1) & 1
            g_fl[lb].wait()
            s_fl[lb] = pltpu.async_copy(rows[lb], acc.at[didx.at[ROUND - 1]],
                                        ssem[lb], add=True)
            g_fl = [None, None]
        for h in s_fl:
            if h is not None:
                h.wait()
        plsc.subcore_barrier()

        st_fl = [None, None]
        for j in range(RCH_SUB):
            b = j & 1
            if st_fl[b] is not None:
                st_fl[b].wait()
            pltpu.async_copy(acc.at[pl.ds(row0 + j * RCH, RCH)], rows[b],
                             gsem[b]).wait()
            st_fl[b] = pltpu.async_copy(
                rows[b], out_hbm.at[c, pl.ds(row0 + j * RCH, RCH)], ssem[b])
        for h in st_fl:
            h.wait()

    return sc_aggregate


R = 1024


def _dinv_from_cnt(cnt_blk):
    return lax.rsqrt(cnt_blk[0, :, :1])


def _mm_first_body(x_ref, w_ref, cnt_ref, o_ref):
    dinv = _dinv_from_cnt(cnt_ref[...])
    g = jnp.dot(x_ref[...], w_ref[...],
                preferred_element_type=jnp.float32) * dinv
    o_ref[0] = g[:, :HW]
    o_ref[1] = g[:, HW:]


_mm_first = pl.pallas_call(
    _mm_first_body,
    grid=(N_PAD // R,),
    in_specs=[
        pl.BlockSpec((R, D_IN), lambda i: (i, 0)),
        pl.BlockSpec((D_IN, HID), lambda i: (0, 0)),
        pl.BlockSpec((1, R, HW), lambda i: (0, i, 0)),
    ],
    out_specs=pl.BlockSpec((NC, R, HW), lambda i: (0, i, 0)),
    out_shape=jax.ShapeDtypeStruct((NC, N_PAD, HW), jnp.float32),
)


def _mm_mid_body(a_ref, cnt_ref, b_ref, w_ref, o_ref):
    dinv = _dinv_from_cnt(cnt_ref[...])
    aa = jnp.concatenate([a_ref[0], a_ref[1]], axis=1)
    h = jnp.maximum(aa * dinv + b_ref[...], 0.0)
    g = jnp.dot(h, w_ref[...], preferred_element_type=jnp.float32) * dinv
    o_ref[0] = g[:, :HW]
    o_ref[1] = g[:, HW:]


_mm_mid = pl.pallas_call(
    _mm_mid_body,
    grid=(N_PAD // R,),
    in_specs=[
        pl.BlockSpec((NC, R, HW), lambda i: (0, i, 0)),
        pl.BlockSpec((1, R, HW), lambda i: (0, i, 0)),
        pl.BlockSpec((1, HID), lambda i: (0, 0)),
        pl.BlockSpec((HID, HID), lambda i: (0, 0)),
    ],
    out_specs=pl.BlockSpec((NC, R, HW), lambda i: (0, i, 0)),
    out_shape=jax.ShapeDtypeStruct((NC, N_PAD, HW), jnp.float32),
)


def _scale_mid_body(a_ref, cnt_ref, b_ref, o_ref):
    dinv = _dinv_from_cnt(cnt_ref[...])
    aa = jnp.concatenate([a_ref[0], a_ref[1]], axis=1)
    q = jnp.maximum(aa * dinv + b_ref[...], 0.0) * dinv
    o_ref[0] = q[:, :HW]
    o_ref[1] = q[:, HW:]


_scale_mid = pl.pallas_call(
    _scale_mid_body,
    grid=(N_PAD // R,),
    in_specs=[
        pl.BlockSpec((NC, R, HW), lambda i: (0, i, 0)),
        pl.BlockSpec((1, R, HW), lambda i: (0, i, 0)),
        pl.BlockSpec((1, HID), lambda i: (0, 0)),
    ],
    out_specs=pl.BlockSpec((NC, R, HW), lambda i: (0, i, 0)),
    out_shape=jax.ShapeDtypeStruct((NC, N_PAD, HW), jnp.float32),
)


def _mm_last_body(a_ref, cnt_ref, b_ref, w_ref, o_ref):
    dinv = _dinv_from_cnt(cnt_ref[...])
    aa = jnp.concatenate([a_ref[0], a_ref[1]], axis=1)
    o_ref[...] = jnp.dot(aa * dinv, w_ref[...],
                         preferred_element_type=jnp.float32) + b_ref[...]


_mm_last = pl.pallas_call(
    _mm_last_body,
    grid=(N_PAD // R,),
    in_specs=[
        pl.BlockSpec((NC, R, HW), lambda i: (0, i, 0)),
        pl.BlockSpec((1, R, HW), lambda i: (0, i, 0)),
        pl.BlockSpec((1, NCLS), lambda i: (0, 0)),
        pl.BlockSpec((HID, NCLS), lambda i: (0, 0)),
    ],
    out_specs=pl.BlockSpec((R, NCLS), lambda i: (i, 0)),
    out_shape=jax.ShapeDtypeStruct((N, NCLS), jnp.float32),
)


def kernel(x, edge_index, W1, b1, W2, b2, W3, b3):
    src = edge_index[0].astype(jnp.int32)
    dst = edge_index[1].astype(jnp.int32)
    pad = E_PAD - E
    src = jnp.concatenate([src, jnp.zeros((pad,), jnp.int32)])
    dst = jnp.concatenate([dst, jnp.full((pad,), N, jnp.int32)])
    srcs2 = jnp.concatenate([src, src + N_PAD]).reshape(-1, K_EDGE)
    dst2 = dst.reshape(-1, K_EDGE)

    agg = _get_sc_aggregate()

    cnt = agg(jnp.ones((NC * N_PAD, HW), jnp.float32), srcs2, dst2)

    g1 = _mm_first(x, W1, cnt)
    a1 = agg(g1.reshape(NC * N_PAD, HW), srcs2, dst2)

    g2 = _mm_mid(a1, cnt, b1.reshape(1, HID), W2)
    a2 = agg(g2.reshape(NC * N_PAD, HW), srcs2, dst2)

    q = _scale_mid(a2, cnt, b2.reshape(1, HID))
    a3 = agg(q.reshape(NC * N_PAD, HW), srcs2, dst2)

    return _mm_last(a3, cnt, b3.reshape(1, NCLS), W3)
